# Initial kernel scaffold; baseline (speedup 1.0000x reference)
import jax
import jax.numpy as jnp
from jax.experimental import pallas as pl


def kernel(z, pool_vectors, lambda_sharp, temperature, W_Q, W_K, aspect_logits, tau):
    raise NotImplementedError("write your pallas kernel here")


# speedup vs baseline: 2.7372x; 2.7372x over previous
import functools

import jax
import jax.numpy as jnp
from jax import lax
from jax.experimental import pallas as pl
from jax.experimental.pallas import tpu as pltpu

NEG = -1000000000.0
K_TOP = 16


def _keys_body(pool_ref, wk_ref, keys_ref, knorm_ref, *, dk):
    kb = jnp.dot(pool_ref[...], wk_ref[...], preferred_element_type=jnp.float32)
    k1 = kb[:, :dk]
    k2 = kb[:, dk:]
    n1 = jnp.sqrt(jnp.sum(k1 * k1, axis=1, keepdims=True)) + 1e-08
    n2 = jnp.sqrt(jnp.sum(k2 * k2, axis=1, keepdims=True)) + 1e-08
    keys_ref[...] = kb.reshape(keys_ref.shape)
    knorm_ref[...] = jnp.concatenate([k1 / n1, k2 / n2], axis=1)


def _keys_call(pool, wk_f, *, bn, dk):
    n, d = pool.shape
    c = wk_f.shape[1]
    grid = (n // bn,)
    return pl.pallas_call(
        functools.partial(_keys_body, dk=dk),
        grid=grid,
        in_specs=[
            pl.BlockSpec((bn, d), lambda i: (i, 0)),
            pl.BlockSpec((d, c), lambda i: (0, 0)),
        ],
        out_specs=[
            pl.BlockSpec((bn, 2, dk), lambda i: (i, 0, 0)),
            pl.BlockSpec((bn, c), lambda i: (i, 0)),
        ],
        out_shape=[
            jax.ShapeDtypeStruct((n, 2, dk), jnp.float32),
            jax.ShapeDtypeStruct((n, c), jnp.float32),
        ],
    )(pool, wk_f)


def _scores_body(z_ref, wq_ref, al_ref, knorm_ref, out_ref, *, dk):
    q = jnp.dot(z_ref[...], wq_ref[...], preferred_element_type=jnp.float32)
    q1 = q[:, :dk]
    q2 = q[:, dk:]
    n1 = jnp.sqrt(jnp.sum(q1 * q1, axis=1, keepdims=True)) + 1e-08
    n2 = jnp.sqrt(jnp.sum(q2 * q2, axis=1, keepdims=True)) + 1e-08
    al = al_ref[...]
    m = jnp.max(al, axis=1, keepdims=True)
    e = jnp.exp(al - m)
    w = e / jnp.sum(e, axis=1, keepdims=True)
    kn = knorm_ref[...]
    s0 = lax.dot_general(q1 / n1, kn[:, :dk], (((1,), (1,)), ((), ())),
                         preferred_element_type=jnp.float32)
    s1 = lax.dot_general(q2 / n2, kn[:, dk:], (((1,), (1,)), ((), ())),
                         preferred_element_type=jnp.float32)
    bf = lambda x: x.astype(jnp.bfloat16).astype(jnp.float32)
    out_ref[...] = bf(s0) * bf(w[0, 0]) + bf(s1) * bf(w[0, 1])


def _scores_call(z, wq_f, al2, knorm, *, bb, bn, dk):
    b, da = z.shape
    n, c = knorm.shape
    grid = (b // bb, n // bn)
    return pl.pallas_call(
        functools.partial(_scores_body, dk=dk),
        grid=grid,
        in_specs=[
            pl.BlockSpec((bb, da), lambda i, j: (i, 0)),
            pl.BlockSpec((da, c), lambda i, j: (0, 0)),
            pl.BlockSpec((1, 2), lambda i, j: (0, 0)),
            pl.BlockSpec((bn, c), lambda i, j: (j, 0)),
        ],
        out_specs=pl.BlockSpec((bb, bn), lambda i, j: (i, j)),
        out_shape=jax.ShapeDtypeStruct((b, n), jnp.float32),
    )(z, wq_f, al2, knorm)


def _topk_body(s_ref, lam_ref, tau_ref, temp_ref, alpha_ref, sout_ref):
    s = s_ref[...]
    rows, n = s.shape
    work = s
    iot = lax.broadcasted_iota(jnp.int32, (rows, n), 1)
    mask = jnp.zeros((rows, n), dtype=jnp.bool_)
    for _ in range(K_TOP):
        rmax = jnp.max(work, axis=1, keepdims=True)
        elig = work == rmax
        first = jnp.min(jnp.where(elig, iot, n), axis=1, keepdims=True)
        kill = iot == first
        mask = mask | kill
        work = jnp.where(kill, -jnp.inf, work)
    sout = jnp.where(mask, s, NEG)
    lam = lam_ref[0, 0]
    tau = tau_ref[0, 0]
    temp = temp_ref[0, 0]
    g = 1.0 / (1.0 + jnp.exp(-lam * (sout - tau)))
    araw = g * jnp.exp(sout / temp)
    denom = jnp.sum(araw, axis=1, keepdims=True) + 1e-08
    alpha_ref[...] = araw / denom
    sout_ref[...] = sout


def _topk_call(scores, lam, tau, temp, *, bb):
    b, n = scores.shape
    grid = (b // bb,)
    return pl.pallas_call(
        _topk_body,
        grid=grid,
        in_specs=[
            pl.BlockSpec((bb, n), lambda i: (i, 0)),
            pl.BlockSpec((1, 1), lambda i: (0, 0)),
            pl.BlockSpec((1, 1), lambda i: (0, 0)),
            pl.BlockSpec((1, 1), lambda i: (0, 0)),
        ],
        out_specs=[
            pl.BlockSpec((bb, n), lambda i: (i, 0)),
            pl.BlockSpec((bb, n), lambda i: (i, 0)),
        ],
        out_shape=[
            jax.ShapeDtypeStruct((b, n), jnp.float32),
            jax.ShapeDtypeStruct((b, n), jnp.float32),
        ],
    )(scores, lam, tau, temp)


def kernel(z, pool_vectors, lambda_sharp, temperature, W_Q, W_K,
           aspect_logits, tau):
    s, da, dk = W_Q.shape
    n, d = pool_vectors.shape
    b = z.shape[0]
    wq_f = jnp.transpose(W_Q, (1, 0, 2)).reshape(da, s * dk)
    wk_f = jnp.transpose(W_K, (1, 0, 2)).reshape(d, s * dk)
    al2 = aspect_logits.reshape(1, s)

    keys3, knorm = _keys_call(pool_vectors, wk_f, bn=min(512, n), dk=dk)
    scores = _scores_call(z, wq_f, al2, knorm,
                          bb=min(256, b), bn=min(8192, n), dk=dk)
    alpha, sout = _topk_call(
        scores,
        lambda_sharp.reshape(1, 1), tau.reshape(1, 1),
        temperature.reshape(1, 1), bb=min(64, b))
    return (alpha, sout, keys3)
